# baseline (speedup 1.0000x reference)
import functools

import jax
import jax.numpy as jnp
from jax import lax
from jax.experimental import pallas as pl
from jax.experimental.pallas import tpu as pltpu
from jax.experimental.pallas import tpu_sc as plsc

_N = 10000
_E = 320000
_H = 128
_DT = 0.1
_EPS_OT = 0.001
_CG_ITERS = 20
_Q = 3

_NC = 2
_NS = 16
_NTILES = _NC * _NS
_EPT = _E // _NTILES
_CH = 80
_NCH = _EPT // _CH
_RPT = _N // _NS
_ZR = 125

_mesh = plsc.VectorSubcoreMesh(core_axis_name="c", subcore_axis_name="s")
_sc_params = pltpu.CompilerParams(needs_layout_passes=False)


def _zero16():
    return jnp.zeros((16,), jnp.float32)


_NPAD = 10240


@functools.partial(
    pl.kernel,
    out_type=(
        jax.ShapeDtypeStruct((_E,), jnp.float32),
        jax.ShapeDtypeStruct((_NC * _N,), jnp.float32),
        jax.ShapeDtypeStruct((_NC * _N,), jnp.float32),
    ),
    mesh=_mesh,
    scratch_types=[
        pltpu.VMEM_SHARED((_NPAD,), jnp.float32),
        pltpu.VMEM_SHARED((_NPAD,), jnp.float32),
        pltpu.VMEM((_CH,), jnp.int32),
        pltpu.VMEM((_CH,), jnp.int32),
        pltpu.VMEM((_CH, _H), jnp.float32),
        pltpu.VMEM((_CH, _H), jnp.float32),
        pltpu.VMEM((_CH,), jnp.float32),
        pltpu.VMEM((_CH,), jnp.float32),
        pltpu.VMEM((128,), jnp.float32),
        pltpu.VMEM((_NPAD // _NS,), jnp.float32),
        pltpu.SemaphoreType.DMA,
        pltpu.SemaphoreType.DMA,
    ],
    compiler_params=_sc_params,
)
def _edge_kernel(hw_hbm, row_hbm, col_hbm, w2_hbm, dparts_hbm, gparts_hbm,
                 diag_sh, deg_sh, rowi, coli, ag, bg, w2loc, wmloc, z128,
                 stg, sem1, sem2):
    c = lax.axis_index("c")
    s = lax.axis_index("s")
    tid = c * _NS + s
    base = tid * _EPT

    for j in range(128 // 16):
        z128[pl.ds(j * 16, 16)] = _zero16()
    zlen = _NPAD // _NS
    for k in range(zlen // 128):
        sl = pl.ds(s * zlen + k * 128, 128)
        pltpu.sync_copy(z128, diag_sh.at[sl])
        pltpu.sync_copy(z128, deg_sh.at[sl])
    plsc.subcore_barrier()

    def chunk(ci, _):
        off = base + ci * _CH
        pltpu.sync_copy(row_hbm.at[pl.ds(off, _CH)], rowi)
        pltpu.sync_copy(col_hbm.at[pl.ds(off, _CH)], coli)
        cp1 = pltpu.async_copy(hw_hbm.at[rowi], ag, sem1)
        cp2 = pltpu.async_copy(hw_hbm.at[coli], bg, sem2)
        cp1.wait()
        cp2.wait()

        lanes = lax.iota(jnp.int32, 16)
        for g in range(_CH // 16):
            def edge16(k, carry):
                cvec, avec = carry
                e = g * 16 + k
                accd = _zero16()
                acca = _zero16()
                for j in range(_H // 16):
                    sl = pl.ds(j * 16, 16)
                    a = ag[e, sl]
                    b = bg[e, sl]
                    d = a - b
                    accd = accd + d * d
                    acca = acca + a * a
                m = lanes == k
                cvec = jnp.where(m, jnp.sum(accd), cvec)
                avec = jnp.where(m, jnp.sum(acca), avec)
                return cvec, avec

            cvec, avec = lax.fori_loop(0, 16, edge16, (_zero16(), _zero16()))
            cd = cvec * (1.0 / _H)
            ca = avec * (1.0 / _H)
            t = jnp.exp(cd * (-1.0 / _EPS_OT))
            p0 = jnp.clip(t, 0.001, 1.0)
            ps = jnp.clip((p0 + 1e-12) * t, 0.001, 1.0)
            w = 0.7 * p0 + 0.3 * ps
            w2 = w * w * ca
            sl = pl.ds(g * 16, 16)
            w2loc[sl] = w2
            wmloc[sl] = jnp.where(rowi[sl] != coli[sl], w2, 0.0)

        pltpu.sync_copy(w2loc, w2_hbm.at[pl.ds(off, _CH)])
        pltpu.sync_copy(w2loc, diag_sh.at[rowi], add=True)
        pltpu.sync_copy(w2loc, diag_sh.at[coli], add=True)
        pltpu.sync_copy(wmloc, deg_sh.at[rowi], add=True)
        pltpu.sync_copy(wmloc, deg_sh.at[coli], add=True)
        return 0

    lax.fori_loop(0, _NCH, chunk, 0)
    plsc.subcore_barrier()

    zl = _NPAD // _NS
    base_o = s * zl
    tail = _N - (_NS - 1) * zl

    @pl.when(s < _NS - 1)
    def _():
        pltpu.sync_copy(diag_sh.at[pl.ds(base_o, zl)], stg)
        pltpu.sync_copy(stg, dparts_hbm.at[pl.ds(c * _N + base_o, zl)])
        pltpu.sync_copy(deg_sh.at[pl.ds(base_o, zl)], stg)
        pltpu.sync_copy(stg, gparts_hbm.at[pl.ds(c * _N + base_o, zl)])

    @pl.when(s == _NS - 1)
    def _():
        pltpu.sync_copy(diag_sh.at[pl.ds(base_o, tail)], stg.at[pl.ds(0, tail)])
        pltpu.sync_copy(stg.at[pl.ds(0, tail)],
                        dparts_hbm.at[pl.ds(c * _N + base_o, tail)])
        pltpu.sync_copy(deg_sh.at[pl.ds(base_o, tail)], stg.at[pl.ds(0, tail)])
        pltpu.sync_copy(stg.at[pl.ds(0, tail)],
                        gparts_hbm.at[pl.ds(c * _N + base_o, tail)])


@functools.partial(
    pl.kernel,
    out_type=jax.ShapeDtypeStruct((_NC, _N, _H), jnp.float32),
    mesh=_mesh,
    scratch_types=(
        [pltpu.VMEM_SHARED((_N, _H), jnp.float32)]
        + [pltpu.VMEM((_CH,), jnp.int32)] * 8
        + [pltpu.VMEM((_CH,), jnp.float32)] * 4
        + [pltpu.VMEM((_CH, _H), jnp.float32)] * 4
        + [pltpu.SemaphoreType.DMA] * 10
    ),
    compiler_params=_sc_params,
)
def _spmm_kernel(m_hbm, row_hbm, col_hbm, w2_hbm, out_hbm,
                 acc_sh, rq0, rq1, rq2, rq3, cq0, cq1, cq2, cq3,
                 wq0, wq1, wq2, wq3, mr0, mr1, mc0, mc1,
                 gr0, gr1, gc0, gc1, sr0, sr1, sc0, sc1, si0, si1):
    c = lax.axis_index("c")
    s = lax.axis_index("s")
    tid = c * _NS + s
    base = tid * _EPT
    rq = (rq0, rq1, rq2, rq3)
    cq = (cq0, cq1, cq2, cq3)
    wq = (wq0, wq1, wq2, wq3)
    mrs = (mr0, mr1)
    mcs = (mc0, mc1)
    gsr = (gr0, gr1)
    gsc = (gc0, gc1)
    ssr = (sr0, sr1)
    ssc = (sc0, sc1)
    si = (si0, si1)

    rbase = s * 640

    def zb(r, _):
        for j in range(_H // 16):
            mr0[r, pl.ds(j * 16, 16)] = _zero16()
        return 0

    lax.fori_loop(0, _CH, zb, 0)
    nfull = 640 // _CH

    @pl.when(s < _NS - 1)
    def _():
        for k in range(nfull):
            pltpu.sync_copy(mr0, acc_sh.at[pl.ds(rbase + k * _CH, _CH)])

    @pl.when(s == _NS - 1)
    def _():
        for k in range(400 // _CH):
            pltpu.sync_copy(mr0, acc_sh.at[pl.ds(rbase + k * _CH, _CH)])

    plsc.subcore_barrier()

    def idx_start(ci, q, b):
        off = pl.ds(base + ci * _CH, _CH)
        pltpu.async_copy(row_hbm.at[off], rq[q], si[b])
        pltpu.async_copy(col_hbm.at[off], cq[q], si[b])
        pltpu.async_copy(w2_hbm.at[off], wq[q], si[b])

    def idx_wait(q, b):
        off = pl.ds(0, _CH)
        pltpu.make_async_copy(row_hbm.at[off], rq[q], si[b]).wait()
        pltpu.make_async_copy(col_hbm.at[off], cq[q], si[b]).wait()
        pltpu.make_async_copy(w2_hbm.at[off], wq[q], si[b]).wait()

    def gather_start(q, b):
        pltpu.async_copy(m_hbm.at[rq[q]], mrs[b], gsr[b])
        pltpu.async_copy(m_hbm.at[cq[q]], mcs[b], gsc[b])

    def gather_wait(b):
        pltpu.make_async_copy(m_hbm.at[rq[0]], mrs[b], gsr[b]).wait()
        pltpu.make_async_copy(m_hbm.at[cq[0]], mcs[b], gsc[b]).wait()

    def scatter_start(q, b):
        pltpu.async_copy(mcs[b], acc_sh.at[rq[q]], ssr[b], add=True)
        pltpu.async_copy(mrs[b], acc_sh.at[cq[q]], ssc[b], add=True)

    def scatter_wait(b):
        pltpu.make_async_copy(mcs[b], acc_sh.at[rq[0]], ssr[b]).wait()
        pltpu.make_async_copy(mrs[b], acc_sh.at[cq[0]], ssc[b]).wait()

    def scale(q, b):
        mr = mrs[b]
        mc = mcs[b]

        @plsc.parallel_loop(0, _CH // 16, 1)
        def _(g):
            wv = wq[q][pl.ds(g * 16, 16)]
            for k in range(16):
                e = g * 16 + k
                w = wv[k]
                for j in range(_H // 16):
                    sl = pl.ds(j * 16, 16)
                    mr[e, sl] = mr[e, sl] * w
                    mc[e, sl] = mc[e, sl] * w

    pltpu.sync_copy(row_hbm.at[pl.ds(base, _CH)], rq0)
    pltpu.sync_copy(col_hbm.at[pl.ds(base, _CH)], cq0)
    pltpu.sync_copy(w2_hbm.at[pl.ds(base, _CH)], wq0)
    idx_start(1, 1, 1)
    gather_start(0, 0)

    def quad(k, _):
        for q in range(4):
            cur = 4 * k + q
            b = q % 2
            if q < 3:
                idx_start(cur + 2, (q + 2) % 4, b)
            else:
                @pl.when(k < (_NCH - 1) // 4 - 1)
                def _():
                    idx_start(cur + 2, (q + 2) % 4, b)
            idx_wait((q + 1) % 4, 1 - b)
            if q > 0:
                scatter_wait(1 - b)
            else:
                @pl.when(k > 0)
                def _():
                    scatter_wait(1 - b)
            gather_start((q + 1) % 4, 1 - b)
            gather_wait(b)
            scale(q, b)
            scatter_start(q, b)
        return 0

    lax.fori_loop(0, (_NCH - 1) // 4, quad, 0)
    scatter_wait(1)
    gather_wait(0)
    scale(0, 0)
    scatter_start(0, 0)
    scatter_wait(0)
    plsc.subcore_barrier()

    @pl.when(s < _NS - 1)
    def _():
        for k in range(nfull):
            sl = pl.ds(rbase + k * _CH, _CH)
            pltpu.sync_copy(acc_sh.at[sl], mr0)
            pltpu.sync_copy(mr0, out_hbm.at[c, sl])

    @pl.when(s == _NS - 1)
    def _():
        for k in range(400 // _CH):
            sl = pl.ds(rbase + k * _CH, _CH)
            pltpu.sync_copy(acc_sh.at[sl], mr0)
            pltpu.sync_copy(mr0, out_hbm.at[c, sl])


def _encode_body(x_ref, win_ref, bin_ref, lng_ref, lnb_ref, ws_ref,
                 h_ref, hw_ref):
    z = jnp.dot(x_ref[...], win_ref[...],
                preferred_element_type=jnp.float32) + bin_ref[...]
    mu = jnp.mean(z, axis=-1, keepdims=True)
    var = jnp.mean((z - mu) ** 2, axis=-1, keepdims=True)
    z = (z - mu) / jnp.sqrt(var + 1e-5) * lng_ref[...] + lnb_ref[...]
    h = jax.nn.sigmoid(z)
    h_ref[...] = h
    hw_ref[...] = jnp.dot(h, ws_ref[...], preferred_element_type=jnp.float32)


def _encode(x, w_in, b_in, ln_g, ln_b, w_sheaf):
    blk = 1000
    grid = _N // blk
    return pl.pallas_call(
        _encode_body,
        grid=(grid,),
        in_specs=[
            pl.BlockSpec((blk, _H), lambda i: (i, 0)),
            pl.BlockSpec((_H, _H), lambda i: (0, 0)),
            pl.BlockSpec((1, _H), lambda i: (0, 0)),
            pl.BlockSpec((1, _H), lambda i: (0, 0)),
            pl.BlockSpec((1, _H), lambda i: (0, 0)),
            pl.BlockSpec((_H, _H), lambda i: (0, 0)),
        ],
        out_specs=[
            pl.BlockSpec((blk, _H), lambda i: (i, 0)),
            pl.BlockSpec((blk, _H), lambda i: (i, 0)),
        ],
        out_shape=[
            jax.ShapeDtypeStruct((_N, _H), jnp.float32),
            jax.ShapeDtypeStruct((_N, _H), jnp.float32),
        ],
    )(x, w_in, b_in, ln_g, ln_b, w_sheaf)


def _prep_body(dparts_ref, gparts_ref, h_ref,
               diag_ref, isd_ref, rs0_ref, u0_ref):
    diag = jnp.sum(dparts_ref[...], axis=-1, keepdims=True)
    deg = jnp.sum(gparts_ref[...], axis=-1, keepdims=True)
    deg = jnp.maximum(deg, 1e-8)
    isd = 1.0 / jnp.sqrt(deg)
    diag_ref[...] = diag
    isd_ref[...] = isd
    h = h_ref[...]
    rs0_ref[...] = jnp.sum(h * h, axis=0, keepdims=True)
    u0_ref[...] = isd * h


def _prep(dparts_t, gparts_t, h):
    return pl.pallas_call(
        _prep_body,
        out_shape=[
            jax.ShapeDtypeStruct((_N, 1), jnp.float32),
            jax.ShapeDtypeStruct((_N, 1), jnp.float32),
            jax.ShapeDtypeStruct((1, _H), jnp.float32),
            jax.ShapeDtypeStruct((_N, _H), jnp.float32),
        ],
    )(dparts_t, gparts_t, h)


def _cg_body(p_ref, x_ref, r_ref, rs_ref, diag_ref, acc_ref,
             pn_ref, xn_ref, rn_ref, rsn_ref):
    P = p_ref[...]
    dg = diag_ref[...]
    ap = P + _DT * (dg * P - acc_ref[0] - acc_ref[1])
    denom = jnp.sum(P * ap, axis=0, keepdims=True) + 1e-16
    rs = rs_ref[...]
    alpha = rs / denom
    xn_ref[...] = x_ref[...] + P * alpha
    rn = r_ref[...] - ap * alpha
    rn_ref[...] = rn
    rsn = jnp.sum(rn * rn, axis=0, keepdims=True)
    rsn_ref[...] = rsn
    beta = rsn / (rs + 1e-16)
    pn_ref[...] = rn + P * beta


def _cg_step(P, X, R, rs, diag, acc):
    return pl.pallas_call(
        _cg_body,
        out_shape=[
            jax.ShapeDtypeStruct((_N, _H), jnp.float32),
            jax.ShapeDtypeStruct((_N, _H), jnp.float32),
            jax.ShapeDtypeStruct((_N, _H), jnp.float32),
            jax.ShapeDtypeStruct((1, _H), jnp.float32),
        ],
    )(P, X, R, rs, diag, acc)


def _cheby_body(t_ref, tm_ref, u_ref, acc_ref, diag_ref, isd_ref,
                a_ref, b_ref, tn_ref, un_ref):
    lap = diag_ref[...] * u_ref[...] - acc_ref[0] - acc_ref[1]
    tl = t_ref[...] + isd_ref[...] * lap
    tn = a_ref[...] * tl + b_ref[...] * tm_ref[...]
    tn_ref[...] = tn
    un_ref[...] = isd_ref[...] * tn


def _cheby_step(t, tm, u, acc, diag, isd, a, b):
    af = jnp.full((1, 1), a, jnp.float32)
    bf = jnp.full((1, 1), b, jnp.float32)
    return pl.pallas_call(
        _cheby_body,
        out_shape=[
            jax.ShapeDtypeStruct((_N, _H), jnp.float32),
            jax.ShapeDtypeStruct((_N, _H), jnp.float32),
        ],
    )(t, tm, u, acc, diag, isd, af, bf)


def _head_body(h_ref, x_ref, t1_ref, t2_ref, t3_ref, g_ref, asv_ref, aaf_ref,
               w1_ref, b1_ref, w2_ref, b2_ref, out_ref):
    a = jax.nn.softmax(g_ref[...], axis=-1)
    h = h_ref[...]
    hafm = (a[:, 0:1] * h + a[:, 1:2] * t1_ref[...] + a[:, 2:3] * t2_ref[...]
            + a[:, 3:4] * t3_ref[...])
    sv = jax.nn.sigmoid(asv_ref[...])
    af = jax.nn.sigmoid(aaf_ref[...])
    fused = h + sv * x_ref[...] + af * hafm
    hid = jnp.maximum(
        jnp.dot(fused, w1_ref[...], preferred_element_type=jnp.float32)
        + b1_ref[...], 0.0)
    out_ref[...] = (jnp.dot(hid, w2_ref[...], preferred_element_type=jnp.float32)
                    + b2_ref[...])


def _head(h, X, t1, t2, t3, gamma, asv, aaf, w1, b1, w2, b2, nclass):
    blk = 1000
    grid = _N // blk
    row_spec = pl.BlockSpec((blk, _H), lambda i: (i, 0))
    full = lambda shape: pl.BlockSpec(shape, lambda i: (0, 0))
    return pl.pallas_call(
        _head_body,
        grid=(grid,),
        in_specs=[
            row_spec, row_spec, row_spec, row_spec, row_spec,
            full((1, _Q + 1)), full((1, 1)), full((1, 1)),
            full((_H, _H)), full((1, _H)),
            full((_H, nclass)), full((1, nclass)),
        ],
        out_specs=pl.BlockSpec((blk, nclass), lambda i: (i, 0)),
        out_shape=jax.ShapeDtypeStruct((_N, nclass), jnp.float32),
    )(h, X, t1, t2, t3, gamma, asv, aaf, w1, b1, w2, b2)


def kernel(x, edge_index, W_in, b_in, ln_g, ln_b, W_sheaf, gamma_q,
           alpha_svr, alpha_afm, W1, b1, W2, b2):
    nclass = W2.shape[1]
    row = edge_index[0]
    col = edge_index[1]

    h, hw = _encode(x, W_in, b_in.reshape(1, _H), ln_g.reshape(1, _H),
                    ln_b.reshape(1, _H), W_sheaf)

    w2e, dparts, gparts = _edge_kernel(hw, row, col)
    diag, isd, rs0, u0 = _prep(dparts.reshape(_NC, _N).T,
                               gparts.reshape(_NC, _N).T, h)

    def spmv(M):
        return _spmm_kernel(M, row, col, w2e)

    X = jnp.zeros((_N, _H), jnp.float32)
    P = h
    R = h
    rs = rs0
    for _ in range(_CG_ITERS):
        acc = spmv(P)
        P, X, R, rs = _cg_step(P, X, R, rs, diag, acc)

    acc = spmv(u0)
    t1, u1 = _cheby_step(h, h, u0, acc, diag, isd, 1.0, 0.0)
    acc = spmv(u1)
    t2, u2 = _cheby_step(t1, h, u1, acc, diag, isd, 2.0, -1.0)
    acc = spmv(u2)
    t3, _ = _cheby_step(t2, t1, u2, acc, diag, isd, 2.0, -1.0)

    return _head(h, X, t1, t2, t3, gamma_q.reshape(1, _Q + 1),
                 alpha_svr.reshape(1, 1), alpha_afm.reshape(1, 1),
                 W1, b1.reshape(1, _H), W2, b2.reshape(1, nclass), nclass)


# speedup vs baseline: 1.3689x; 1.3689x over previous
import functools

import jax
import jax.numpy as jnp
from jax import lax
from jax.experimental import pallas as pl
from jax.experimental.pallas import tpu as pltpu
from jax.experimental.pallas import tpu_sc as plsc

_N = 10000
_E = 320000
_H = 128
_DT = 0.1
_EPS_OT = 0.001
_CG_ITERS = 20
_Q = 3

_NC = 2
_NS = 16
_NTILES = _NC * _NS
_EPT = _E // _NTILES
_CH = 80
_NCH = _EPT // _CH
_RPT = _N // _NS
_ZR = 125

_mesh = plsc.VectorSubcoreMesh(core_axis_name="c", subcore_axis_name="s")
_sc_params = pltpu.CompilerParams(needs_layout_passes=False)


def _zero16():
    return jnp.zeros((16,), jnp.float32)


_NPAD = 10240


@functools.partial(
    pl.kernel,
    out_type=(
        jax.ShapeDtypeStruct((_E,), jnp.float32),
        jax.ShapeDtypeStruct((_NC * _N,), jnp.float32),
        jax.ShapeDtypeStruct((_NC * _N,), jnp.float32),
    ),
    mesh=_mesh,
    scratch_types=[
        pltpu.VMEM_SHARED((_NPAD,), jnp.float32),
        pltpu.VMEM_SHARED((_NPAD,), jnp.float32),
        pltpu.VMEM((_CH,), jnp.int32),
        pltpu.VMEM((_CH,), jnp.int32),
        pltpu.VMEM((_CH, _H), jnp.float32),
        pltpu.VMEM((_CH, _H), jnp.float32),
        pltpu.VMEM((_CH,), jnp.float32),
        pltpu.VMEM((_CH,), jnp.float32),
        pltpu.VMEM((128,), jnp.float32),
        pltpu.VMEM((_NPAD // _NS,), jnp.float32),
        pltpu.SemaphoreType.DMA,
        pltpu.SemaphoreType.DMA,
    ],
    compiler_params=_sc_params,
)
def _edge_kernel(hw_hbm, row_hbm, col_hbm, w2_hbm, dparts_hbm, gparts_hbm,
                 diag_sh, deg_sh, rowi, coli, ag, bg, w2loc, wmloc, z128,
                 stg, sem1, sem2):
    c = lax.axis_index("c")
    s = lax.axis_index("s")
    tid = c * _NS + s
    base = tid * _EPT

    for j in range(128 // 16):
        z128[pl.ds(j * 16, 16)] = _zero16()
    zlen = _NPAD // _NS
    for k in range(zlen // 128):
        sl = pl.ds(s * zlen + k * 128, 128)
        pltpu.sync_copy(z128, diag_sh.at[sl])
        pltpu.sync_copy(z128, deg_sh.at[sl])
    plsc.subcore_barrier()

    def chunk(ci, _):
        off = base + ci * _CH
        pltpu.sync_copy(row_hbm.at[pl.ds(off, _CH)], rowi)
        pltpu.sync_copy(col_hbm.at[pl.ds(off, _CH)], coli)
        cp1 = pltpu.async_copy(hw_hbm.at[rowi], ag, sem1)
        cp2 = pltpu.async_copy(hw_hbm.at[coli], bg, sem2)
        cp1.wait()
        cp2.wait()

        lanes = lax.iota(jnp.int32, 16)
        for g in range(_CH // 16):
            def edge16(k, carry):
                cvec, avec = carry
                e = g * 16 + k
                accd = _zero16()
                acca = _zero16()
                for j in range(_H // 16):
                    sl = pl.ds(j * 16, 16)
                    a = ag[e, sl]
                    b = bg[e, sl]
                    d = a - b
                    accd = accd + d * d
                    acca = acca + a * a
                m = lanes == k
                cvec = jnp.where(m, jnp.sum(accd), cvec)
                avec = jnp.where(m, jnp.sum(acca), avec)
                return cvec, avec

            cvec, avec = lax.fori_loop(0, 16, edge16, (_zero16(), _zero16()))
            cd = cvec * (1.0 / _H)
            ca = avec * (1.0 / _H)
            t = jnp.exp(cd * (-1.0 / _EPS_OT))
            p0 = jnp.clip(t, 0.001, 1.0)
            ps = jnp.clip((p0 + 1e-12) * t, 0.001, 1.0)
            w = 0.7 * p0 + 0.3 * ps
            w2 = w * w * ca
            sl = pl.ds(g * 16, 16)
            w2loc[sl] = w2
            wmloc[sl] = jnp.where(rowi[sl] != coli[sl], w2, 0.0)

        pltpu.sync_copy(w2loc, w2_hbm.at[pl.ds(off, _CH)])
        pltpu.sync_copy(w2loc, diag_sh.at[rowi], add=True)
        pltpu.sync_copy(w2loc, diag_sh.at[coli], add=True)
        pltpu.sync_copy(wmloc, deg_sh.at[rowi], add=True)
        pltpu.sync_copy(wmloc, deg_sh.at[coli], add=True)
        return 0

    lax.fori_loop(0, _NCH, chunk, 0)
    plsc.subcore_barrier()

    zl = _NPAD // _NS
    base_o = s * zl
    tail = _N - (_NS - 1) * zl

    @pl.when(s < _NS - 1)
    def _():
        pltpu.sync_copy(diag_sh.at[pl.ds(base_o, zl)], stg)
        pltpu.sync_copy(stg, dparts_hbm.at[pl.ds(c * _N + base_o, zl)])
        pltpu.sync_copy(deg_sh.at[pl.ds(base_o, zl)], stg)
        pltpu.sync_copy(stg, gparts_hbm.at[pl.ds(c * _N + base_o, zl)])

    @pl.when(s == _NS - 1)
    def _():
        pltpu.sync_copy(diag_sh.at[pl.ds(base_o, tail)], stg.at[pl.ds(0, tail)])
        pltpu.sync_copy(stg.at[pl.ds(0, tail)],
                        dparts_hbm.at[pl.ds(c * _N + base_o, tail)])
        pltpu.sync_copy(deg_sh.at[pl.ds(base_o, tail)], stg.at[pl.ds(0, tail)])
        pltpu.sync_copy(stg.at[pl.ds(0, tail)],
                        gparts_hbm.at[pl.ds(c * _N + base_o, tail)])


@functools.partial(
    pl.kernel,
    out_type=jax.ShapeDtypeStruct((_NC, _N, _H), jnp.float32),
    mesh=_mesh,
    scratch_types=(
        [pltpu.VMEM_SHARED((_N, _H), jnp.float32)]
        + [pltpu.VMEM((_CH,), jnp.int32)] * 8
        + [pltpu.VMEM((_CH,), jnp.float32)] * 4
        + [pltpu.VMEM((_CH, _H), jnp.float32)] * 4
        + [pltpu.SemaphoreType.DMA] * 10
    ),
    compiler_params=_sc_params,
)
def _spmm_kernel(m_hbm, row_hbm, col_hbm, w2_hbm, out_hbm,
                 acc_sh, rq0, rq1, rq2, rq3, cq0, cq1, cq2, cq3,
                 wq0, wq1, wq2, wq3, mr0, mr1, mc0, mc1,
                 gr0, gr1, gc0, gc1, sr0, sr1, sc0, sc1, si0, si1):
    c = lax.axis_index("c")
    s = lax.axis_index("s")
    tid = c * _NS + s
    base = tid * _EPT
    rq = (rq0, rq1, rq2, rq3)
    cq = (cq0, cq1, cq2, cq3)
    wq = (wq0, wq1, wq2, wq3)
    mrs = (mr0, mr1)
    mcs = (mc0, mc1)
    gsr = (gr0, gr1)
    gsc = (gc0, gc1)
    ssr = (sr0, sr1)
    ssc = (sc0, sc1)
    si = (si0, si1)

    rbase = s * 640

    def zb(r, _):
        for j in range(_H // 16):
            mr0[r, pl.ds(j * 16, 16)] = _zero16()
        return 0

    lax.fori_loop(0, _CH, zb, 0)
    nfull = 640 // _CH

    @pl.when(s < _NS - 1)
    def _():
        for k in range(nfull):
            pltpu.sync_copy(mr0, acc_sh.at[pl.ds(rbase + k * _CH, _CH)])

    @pl.when(s == _NS - 1)
    def _():
        for k in range(400 // _CH):
            pltpu.sync_copy(mr0, acc_sh.at[pl.ds(rbase + k * _CH, _CH)])

    plsc.subcore_barrier()

    def idx_start(ci, q, b):
        off = pl.ds(base + ci * _CH, _CH)
        pltpu.async_copy(row_hbm.at[off], rq[q], si[b])
        pltpu.async_copy(col_hbm.at[off], cq[q], si[b])
        pltpu.async_copy(w2_hbm.at[off], wq[q], si[b])

    def idx_wait(q, b):
        off = pl.ds(0, _CH)
        pltpu.make_async_copy(row_hbm.at[off], rq[q], si[b]).wait()
        pltpu.make_async_copy(col_hbm.at[off], cq[q], si[b]).wait()
        pltpu.make_async_copy(w2_hbm.at[off], wq[q], si[b]).wait()

    def gather_start(q, b):
        pltpu.async_copy(m_hbm.at[rq[q]], mrs[b], gsr[b])
        pltpu.async_copy(m_hbm.at[cq[q]], mcs[b], gsc[b])

    def gather_wait(b):
        pltpu.make_async_copy(m_hbm.at[rq[0]], mrs[b], gsr[b]).wait()
        pltpu.make_async_copy(m_hbm.at[cq[0]], mcs[b], gsc[b]).wait()

    def scatter_wait(b):
        pltpu.make_async_copy(mcs[b], acc_sh.at[rq[0]], ssr[b]).wait()
        pltpu.make_async_copy(mrs[b], acc_sh.at[cq[0]], ssc[b]).wait()

    def scale_one(buf, q):
        def group(g, _):
            wv = wq[q][pl.ds(g * 16, 16)]
            for k in range(16):
                e = g * 16 + k
                w = wv[k]
                for j in range(_H // 16):
                    sl = pl.ds(j * 16, 16)
                    buf[e, sl] = buf[e, sl] * w
            return 0

        lax.fori_loop(0, _CH // 16, group, 0)

    def scale(q, b):
        scale_one(mcs[b], q)
        pltpu.async_copy(mcs[b], acc_sh.at[rq[q]], ssr[b], add=True)
        scale_one(mrs[b], q)
        pltpu.async_copy(mrs[b], acc_sh.at[cq[q]], ssc[b], add=True)

    pltpu.sync_copy(row_hbm.at[pl.ds(base, _CH)], rq0)
    pltpu.sync_copy(col_hbm.at[pl.ds(base, _CH)], cq0)
    pltpu.sync_copy(w2_hbm.at[pl.ds(base, _CH)], wq0)
    idx_start(1, 1, 1)
    gather_start(0, 0)

    def quad(k, _):
        for q in range(4):
            cur = 4 * k + q
            b = q % 2
            if q < 3:
                idx_start(cur + 2, (q + 2) % 4, b)
            else:
                @pl.when(k < (_NCH - 1) // 4 - 1)
                def _():
                    idx_start(cur + 2, (q + 2) % 4, b)
            idx_wait((q + 1) % 4, 1 - b)
            if q > 0:
                scatter_wait(1 - b)
            else:
                @pl.when(k > 0)
                def _():
                    scatter_wait(1 - b)
            gather_start((q + 1) % 4, 1 - b)
            gather_wait(b)
            scale(q, b)
        return 0

    lax.fori_loop(0, (_NCH - 1) // 4, quad, 0)
    scatter_wait(1)
    gather_wait(0)
    scale(0, 0)
    scatter_wait(0)
    plsc.subcore_barrier()

    @pl.when(s < _NS - 1)
    def _():
        for k in range(nfull):
            sl = pl.ds(rbase + k * _CH, _CH)
            pltpu.sync_copy(acc_sh.at[sl], mr0)
            pltpu.sync_copy(mr0, out_hbm.at[c, sl])

    @pl.when(s == _NS - 1)
    def _():
        for k in range(400 // _CH):
            sl = pl.ds(rbase + k * _CH, _CH)
            pltpu.sync_copy(acc_sh.at[sl], mr0)
            pltpu.sync_copy(mr0, out_hbm.at[c, sl])


def _encode_body(x_ref, win_ref, bin_ref, lng_ref, lnb_ref, ws_ref,
                 h_ref, hw_ref):
    z = jnp.dot(x_ref[...], win_ref[...],
                preferred_element_type=jnp.float32) + bin_ref[...]
    mu = jnp.mean(z, axis=-1, keepdims=True)
    var = jnp.mean((z - mu) ** 2, axis=-1, keepdims=True)
    z = (z - mu) / jnp.sqrt(var + 1e-5) * lng_ref[...] + lnb_ref[...]
    h = jax.nn.sigmoid(z)
    h_ref[...] = h
    hw_ref[...] = jnp.dot(h, ws_ref[...], preferred_element_type=jnp.float32)


def _encode(x, w_in, b_in, ln_g, ln_b, w_sheaf):
    blk = 1000
    grid = _N // blk
    return pl.pallas_call(
        _encode_body,
        grid=(grid,),
        in_specs=[
            pl.BlockSpec((blk, _H), lambda i: (i, 0)),
            pl.BlockSpec((_H, _H), lambda i: (0, 0)),
            pl.BlockSpec((1, _H), lambda i: (0, 0)),
            pl.BlockSpec((1, _H), lambda i: (0, 0)),
            pl.BlockSpec((1, _H), lambda i: (0, 0)),
            pl.BlockSpec((_H, _H), lambda i: (0, 0)),
        ],
        out_specs=[
            pl.BlockSpec((blk, _H), lambda i: (i, 0)),
            pl.BlockSpec((blk, _H), lambda i: (i, 0)),
        ],
        out_shape=[
            jax.ShapeDtypeStruct((_N, _H), jnp.float32),
            jax.ShapeDtypeStruct((_N, _H), jnp.float32),
        ],
    )(x, w_in, b_in, ln_g, ln_b, w_sheaf)


def _prep_body(dparts_ref, gparts_ref, h_ref,
               diag_ref, isd_ref, rs0_ref, u0_ref):
    diag = jnp.sum(dparts_ref[...], axis=-1, keepdims=True)
    deg = jnp.sum(gparts_ref[...], axis=-1, keepdims=True)
    deg = jnp.maximum(deg, 1e-8)
    isd = 1.0 / jnp.sqrt(deg)
    diag_ref[...] = diag
    isd_ref[...] = isd
    h = h_ref[...]
    rs0_ref[...] = jnp.sum(h * h, axis=0, keepdims=True)
    u0_ref[...] = isd * h


def _prep(dparts_t, gparts_t, h):
    return pl.pallas_call(
        _prep_body,
        out_shape=[
            jax.ShapeDtypeStruct((_N, 1), jnp.float32),
            jax.ShapeDtypeStruct((_N, 1), jnp.float32),
            jax.ShapeDtypeStruct((1, _H), jnp.float32),
            jax.ShapeDtypeStruct((_N, _H), jnp.float32),
        ],
    )(dparts_t, gparts_t, h)


def _cg_body(p_ref, x_ref, r_ref, rs_ref, diag_ref, acc_ref,
             pn_ref, xn_ref, rn_ref, rsn_ref):
    P = p_ref[...]
    dg = diag_ref[...]
    ap = P + _DT * (dg * P - acc_ref[0] - acc_ref[1])
    denom = jnp.sum(P * ap, axis=0, keepdims=True) + 1e-16
    rs = rs_ref[...]
    alpha = rs / denom
    xn_ref[...] = x_ref[...] + P * alpha
    rn = r_ref[...] - ap * alpha
    rn_ref[...] = rn
    rsn = jnp.sum(rn * rn, axis=0, keepdims=True)
    rsn_ref[...] = rsn
    beta = rsn / (rs + 1e-16)
    pn_ref[...] = rn + P * beta


def _cg_step(P, X, R, rs, diag, acc):
    return pl.pallas_call(
        _cg_body,
        out_shape=[
            jax.ShapeDtypeStruct((_N, _H), jnp.float32),
            jax.ShapeDtypeStruct((_N, _H), jnp.float32),
            jax.ShapeDtypeStruct((_N, _H), jnp.float32),
            jax.ShapeDtypeStruct((1, _H), jnp.float32),
        ],
    )(P, X, R, rs, diag, acc)


def _cheby_body(t_ref, tm_ref, u_ref, acc_ref, diag_ref, isd_ref,
                a_ref, b_ref, tn_ref, un_ref):
    lap = diag_ref[...] * u_ref[...] - acc_ref[0] - acc_ref[1]
    tl = t_ref[...] + isd_ref[...] * lap
    tn = a_ref[...] * tl + b_ref[...] * tm_ref[...]
    tn_ref[...] = tn
    un_ref[...] = isd_ref[...] * tn


def _cheby_step(t, tm, u, acc, diag, isd, a, b):
    af = jnp.full((1, 1), a, jnp.float32)
    bf = jnp.full((1, 1), b, jnp.float32)
    return pl.pallas_call(
        _cheby_body,
        out_shape=[
            jax.ShapeDtypeStruct((_N, _H), jnp.float32),
            jax.ShapeDtypeStruct((_N, _H), jnp.float32),
        ],
    )(t, tm, u, acc, diag, isd, af, bf)


def _head_body(h_ref, x_ref, t1_ref, t2_ref, t3_ref, g_ref, asv_ref, aaf_ref,
               w1_ref, b1_ref, w2_ref, b2_ref, out_ref):
    a = jax.nn.softmax(g_ref[...], axis=-1)
    h = h_ref[...]
    hafm = (a[:, 0:1] * h + a[:, 1:2] * t1_ref[...] + a[:, 2:3] * t2_ref[...]
            + a[:, 3:4] * t3_ref[...])
    sv = jax.nn.sigmoid(asv_ref[...])
    af = jax.nn.sigmoid(aaf_ref[...])
    fused = h + sv * x_ref[...] + af * hafm
    hid = jnp.maximum(
        jnp.dot(fused, w1_ref[...], preferred_element_type=jnp.float32)
        + b1_ref[...], 0.0)
    out_ref[...] = (jnp.dot(hid, w2_ref[...], preferred_element_type=jnp.float32)
                    + b2_ref[...])


def _head(h, X, t1, t2, t3, gamma, asv, aaf, w1, b1, w2, b2, nclass):
    blk = 1000
    grid = _N // blk
    row_spec = pl.BlockSpec((blk, _H), lambda i: (i, 0))
    full = lambda shape: pl.BlockSpec(shape, lambda i: (0, 0))
    return pl.pallas_call(
        _head_body,
        grid=(grid,),
        in_specs=[
            row_spec, row_spec, row_spec, row_spec, row_spec,
            full((1, _Q + 1)), full((1, 1)), full((1, 1)),
            full((_H, _H)), full((1, _H)),
            full((_H, nclass)), full((1, nclass)),
        ],
        out_specs=pl.BlockSpec((blk, nclass), lambda i: (i, 0)),
        out_shape=jax.ShapeDtypeStruct((_N, nclass), jnp.float32),
    )(h, X, t1, t2, t3, gamma, asv, aaf, w1, b1, w2, b2)


def kernel(x, edge_index, W_in, b_in, ln_g, ln_b, W_sheaf, gamma_q,
           alpha_svr, alpha_afm, W1, b1, W2, b2):
    nclass = W2.shape[1]
    row = edge_index[0]
    col = edge_index[1]

    h, hw = _encode(x, W_in, b_in.reshape(1, _H), ln_g.reshape(1, _H),
                    ln_b.reshape(1, _H), W_sheaf)

    w2e, dparts, gparts = _edge_kernel(hw, row, col)
    diag, isd, rs0, u0 = _prep(dparts.reshape(_NC, _N).T,
                               gparts.reshape(_NC, _N).T, h)

    def spmv(M):
        return _spmm_kernel(M, row, col, w2e)

    X = jnp.zeros((_N, _H), jnp.float32)
    P = h
    R = h
    rs = rs0
    for _ in range(_CG_ITERS):
        acc = spmv(P)
        P, X, R, rs = _cg_step(P, X, R, rs, diag, acc)

    acc = spmv(u0)
    t1, u1 = _cheby_step(h, h, u0, acc, diag, isd, 1.0, 0.0)
    acc = spmv(u1)
    t2, u2 = _cheby_step(t1, h, u1, acc, diag, isd, 2.0, -1.0)
    acc = spmv(u2)
    t3, _ = _cheby_step(t2, t1, u2, acc, diag, isd, 2.0, -1.0)

    return _head(h, X, t1, t2, t3, gamma_q.reshape(1, _Q + 1),
                 alpha_svr.reshape(1, 1), alpha_afm.reshape(1, 1),
                 W1, b1.reshape(1, _H), W2, b2.reshape(1, nclass), nclass)
